# baseline (speedup 1.0000x reference)
import functools

import jax
import jax.numpy as jnp
from jax.experimental import pallas as pl
from jax.experimental.pallas import tpu as pltpu

_N = 10000
_E = 2000
_D = 128
_CH = 400
_NCH = _N // _CH
_DN = (((0,), (0,)), ((), ()))
_DNAT = (((1,), (0,)), ((), ()))
_F32 = jnp.float32


def _hypersage_kernel(x_ref, inc_ref, w1_ref, w2_ref, wlin_ref, blin_ref,
                      out_ref, inc_bf, feat, edge, ehi, deg_row, pooled):
    p = pl.program_id(0)
    i = pl.program_id(1)
    first = i == 0
    bf16 = jnp.bfloat16

    @pl.when(jnp.logical_and(p == 0, first))
    def _():
        edge[...] = jnp.zeros((_E, _D), _F32)
        deg_row[...] = jnp.zeros((1, _E), _F32)

    @pl.when(p == 0)
    def _():
        blk = inc_ref[...]
        bb = blk.astype(bf16)
        inc_bf[pl.ds(i * _CH, _CH), :] = bb
        deg_row[...] += jnp.sum(blk, axis=0, keepdims=True)
        f = x_ref[...]
        hi = (f * f).astype(bf16)
        edge[...] += jax.lax.dot_general(bb, hi, _DN,
                                         preferred_element_type=_F32)

    @pl.when(jnp.logical_and(p == 1, first))
    def _():
        ide = 1.0 / jnp.transpose(deg_row[...])
        ehi[...] = (edge[...] * ide).astype(bf16)

    def inter_edge(W):
        r = pl.ds(i * _CH, _CH)
        inc_c = inc_bf[r, :]
        t = jax.lax.dot_general(inc_c, ehi[...], _DNAT,
                                preferred_element_type=_F32)
        h = jax.lax.dot_general(jnp.sqrt(t), W, _DNAT,
                                preferred_element_type=_F32)
        h = jnp.maximum(h, 0.0)
        norm = jnp.sqrt(jnp.sum(h * h, axis=-1, keepdims=True))
        return h / (norm + 1e-12)

    @pl.when(p == 1)
    def _():
        feat[pl.ds(i * _CH, _CH), :] = inter_edge(w1_ref[...])

    @pl.when(jnp.logical_and(p == 2, first))
    def _():
        edge[...] = jnp.zeros((_E, _D), _F32)

    @pl.when(p == 2)
    def _():
        r = pl.ds(i * _CH, _CH)
        f = feat[r, :]
        hi = (f * f).astype(bf16)
        edge[...] += jax.lax.dot_general(inc_bf[r, :], hi, _DN,
                                         preferred_element_type=_F32)

    @pl.when(jnp.logical_and(p == 3, first))
    def _():
        ide = 1.0 / jnp.transpose(deg_row[...])
        ehi[...] = (edge[...] * ide).astype(bf16)
        pooled[...] = jnp.full((1, _D), -jnp.inf, _F32)

    @pl.when(p == 3)
    def _():
        h = inter_edge(w2_ref[...])
        pooled[...] = jnp.maximum(pooled[...],
                                  jnp.max(h, axis=0, keepdims=True))

    @pl.when(jnp.logical_and(p == 3, i == _NCH - 1))
    def _():
        dn_t = (((1,), (1,)), ((), ()))
        out_ref[...] = (
            jax.lax.dot_general(pooled[...], wlin_ref[...], dn_t,
                                preferred_element_type=_F32)
            + blin_ref[...])


@jax.jit
def kernel(x_0, incidence, W1, W2, Wlin, b_lin):
    grid = (4, _NCH)

    def chunk0_map(p, i):
        return (jnp.where(p == 0, i, 0), 0)

    def const_map(p, i):
        return (0, 0)

    out = pl.pallas_call(
        _hypersage_kernel,
        grid=grid,
        in_specs=[
            pl.BlockSpec((_CH, _D), chunk0_map),
            pl.BlockSpec((_CH, _E), chunk0_map),
            pl.BlockSpec((_D, _D), const_map),
            pl.BlockSpec((_D, _D), const_map),
            pl.BlockSpec((_D, _D), const_map),
            pl.BlockSpec((1, _D), const_map),
        ],
        out_specs=pl.BlockSpec((1, _D), const_map),
        out_shape=jax.ShapeDtypeStruct((1, _D), jnp.float32),
        scratch_shapes=[
            pltpu.VMEM((_N, _E), jnp.bfloat16),
            pltpu.VMEM((_N, _D), _F32),
            pltpu.VMEM((_E, _D), _F32),
            pltpu.VMEM((_E, _D), jnp.bfloat16),
            pltpu.VMEM((1, _E), _F32),
            pltpu.VMEM((1, _D), _F32),
        ],
        compiler_params=pltpu.CompilerParams(
            vmem_limit_bytes=64 * 1024 * 1024,
        ),
    )(x_0, incidence, W1, W2, Wlin, b_lin.reshape(1, -1))
    return out.reshape(-1)
